# baseline (speedup 1.0000x reference)
import functools

import jax
import jax.numpy as jnp
from jax import lax
from jax.experimental import pallas as pl
from jax.experimental.pallas import tpu as pltpu
from jax.experimental.pallas import tpu_sc as plsc

N_NODES = 10000
N_ACC = 10240
NW = 32
CHUNK = 128
NC = 80
E_PAD = NW * NC * CHUNK
NB = 2
BN_TC = 1000


def _make_sc_agg(D):
    mesh = plsc.VectorSubcoreMesh(core_axis_name="c", subcore_axis_name="s")
    NI = 2 * NB

    @functools.partial(
        pl.kernel,
        out_type=jax.ShapeDtypeStruct((2, N_NODES, D), jnp.float32),
        mesh=mesh,
        scratch_types=[
            pltpu.VMEM((NI, CHUNK), jnp.int32),
            pltpu.VMEM((NI, CHUNK), jnp.int32),
            pltpu.VMEM((NB, CHUNK, D), jnp.float32),
            pltpu.VMEM_SHARED((N_ACC, D), jnp.float32),
        ] + [pltpu.SemaphoreType.DMA] * (NI + NI + NB),
        compiler_params=pltpu.CompilerParams(use_tc_tiling_on_sc=False),
    )
    def k(table, src_r, dst_r, out, sidx, didx, rows, acc, *sems):
        sisem = sems[:NI]
        disem = sems[NI:2 * NI]
        gsem = sems[2 * NI:]
        c = lax.axis_index("c")
        s = lax.axis_index("s")
        wid = c * 16 + s
        tab = table.at[c * 4 + s // 4]

        zvec = jnp.zeros((16,), jnp.float32)

        def zrow(i, _):
            for j in range(D // 16):
                rows[0, i, pl.ds(j * 16, 16)] = zvec
            return 0

        lax.fori_loop(0, CHUNK, zrow, 0)
        for z in range(5):
            pltpu.sync_copy(rows.at[0], acc.at[pl.ds(s * 640 + z * CHUNK, CHUNK)])
        plsc.subcore_barrier()

        def fire_idx(j, sl):
            pltpu.async_copy(src_r.at[wid, j], sidx.at[sl], sisem[sl])
            pltpu.async_copy(dst_r.at[wid, j], didx.at[sl], disem[sl])

        def fire_gather(j, sl, rsl):
            pltpu.make_async_copy(
                src_r.at[0, 0], sidx.at[sl], sisem[sl]).wait()
            pltpu.async_copy(tab.at[sidx.at[sl]], rows.at[rsl], gsem[rsl])

        def do_scatter(j, sl, rsl):
            pltpu.make_async_copy(
                tab.at[sidx.at[0]], rows.at[rsl], gsem[rsl]).wait()
            pltpu.make_async_copy(
                dst_r.at[0, 0], didx.at[sl], disem[sl]).wait()
            pltpu.sync_copy(rows.at[rsl], acc.at[didx.at[sl]], add=True)

        for i in range(2 * NB):
            if i >= NB:
                fire_gather(i - NB, (i - NB) % NI, (i - NB) % NB)
            fire_idx(i, i % NI)

        def body(g, _):
            for u in range(2 * NB):
                i = 2 * NB + g * 2 * NB + u
                do_scatter(i - 2 * NB, u, u % NB)
                fire_gather(i - NB, (u + NB) % NI, u % NB)
                fire_idx(i, u)
            return 0

        lax.fori_loop(0, (NC - 2 * NB) // (2 * NB), body, 0)

        for i in range(NC, NC + 2 * NB):
            do_scatter(i - 2 * NB, (i - 2 * NB) % NI, (i - 2 * NB) % NB)
            if i - NB < NC:
                fire_gather(i - NB, (i - NB) % NI, (i - NB) % NB)
        plsc.subcore_barrier()

        for z in range(5):
            r0 = s * 625 + z * 125
            pltpu.sync_copy(acc.at[pl.ds(r0, 125)], rows.at[0, pl.ds(0, 125)])
            pltpu.sync_copy(rows.at[0, pl.ds(0, 125)], out.at[c, pl.ds(r0, 125)])

    return k


def _tc_layer(Sp, cnt, W, b, sc, sh, first):
    D = Sp.shape[-1]
    H = W.shape[0]
    grid = (N_NODES // BN_TC,)

    def body(*refs):
        if first:
            p_ref, w_ref, b_ref, sc_ref, sh_ref, h_ref, c_ref = refs
        else:
            p_ref, c_in_ref, w_ref, b_ref, sc_ref, sh_ref, h_ref = refs
        P = p_ref[0] + p_ref[1]
        if first:
            S = P[:, :128]
            cc = P[:, 128:129]
        else:
            S = P
            cc = c_in_ref[:, 0:1]
        r = jnp.where(cc > 0, 1.0 / jnp.maximum(cc, 1.0), 0.0)
        A = S * r
        Z = lax.dot_general(A, w_ref[...], (((1,), (1,)), ((), ())),
                            preferred_element_type=jnp.float32)
        Z = jnp.where(cc > 0, Z + b_ref[...], 0.0)
        Z = Z * sc_ref[...] + sh_ref[...]
        h = jnp.where(Z > 0, Z, jnp.exp(Z) - 1.0)
        for _r in range(8):
            h_ref[_r] = h
        if first:
            c_ref[...] = jnp.broadcast_to(cc, (BN_TC, 8))

    in_specs = [pl.BlockSpec((2, BN_TC, D), lambda i: (0, i, 0))]
    if not first:
        in_specs.append(pl.BlockSpec((BN_TC, 8), lambda i: (i, 0)))
    in_specs += [
        pl.BlockSpec(W.shape, lambda i: (0, 0)),
        pl.BlockSpec((1, H), lambda i: (0, 0)),
        pl.BlockSpec((1, H), lambda i: (0, 0)),
        pl.BlockSpec((1, H), lambda i: (0, 0)),
    ]
    out_shape = [jax.ShapeDtypeStruct((8, N_NODES, H), jnp.float32)]
    out_specs = [pl.BlockSpec((8, BN_TC, H), lambda i: (0, i, 0))]
    if first:
        out_shape.append(jax.ShapeDtypeStruct((N_NODES, 8), jnp.float32))
        out_specs.append(pl.BlockSpec((BN_TC, 8), lambda i: (i, 0)))

    args = [Sp] if first else [Sp, cnt]
    args += [W, b.reshape(1, H), sc.reshape(1, H), sh.reshape(1, H)]
    res = pl.pallas_call(
        body, grid=grid, in_specs=in_specs, out_specs=out_specs,
        out_shape=out_shape)(*args)
    return res if first else res[0]


def _tc_final(Sp, cnt, W2, b2, sc2, sh2, Wout, bout):
    D = Sp.shape[-1]
    grid = (N_NODES // BN_TC,)

    def body(p_ref, c_ref, w2_ref, b2_ref, sc_ref, sh_ref, wo_ref, bo_ref,
             o_ref):
        P = p_ref[0] + p_ref[1]
        cc = c_ref[:, 0:1]
        r = jnp.where(cc > 0, 1.0 / jnp.maximum(cc, 1.0), 0.0)
        A = P * r
        Z = lax.dot_general(A, w2_ref[...], (((1,), (1,)), ((), ())),
                            preferred_element_type=jnp.float32)
        Z = jnp.where(cc > 0, Z + b2_ref[...], 0.0)
        Z = Z * sc_ref[...] + sh_ref[...]
        h3 = jnp.where(Z > 0, Z, jnp.exp(Z) - 1.0)
        o_ref[...] = lax.dot_general(h3, wo_ref[...], (((1,), (1,)), ((), ())),
                                     preferred_element_type=jnp.float32) + bo_ref[...]

    return pl.pallas_call(
        body, grid=grid,
        in_specs=[
            pl.BlockSpec((2, BN_TC, D), lambda i: (0, i, 0)),
            pl.BlockSpec((BN_TC, 8), lambda i: (i, 0)),
            pl.BlockSpec(W2.shape, lambda i: (0, 0)),
            pl.BlockSpec((1, 256), lambda i: (0, 0)),
            pl.BlockSpec((1, 256), lambda i: (0, 0)),
            pl.BlockSpec((1, 256), lambda i: (0, 0)),
            pl.BlockSpec(Wout.shape, lambda i: (0, 0)),
            pl.BlockSpec((1, 128), lambda i: (0, 0)),
        ],
        out_specs=pl.BlockSpec((BN_TC, 128), lambda i: (i, 0)),
        out_shape=jax.ShapeDtypeStruct((N_NODES, 128), jnp.float32),
    )(Sp, cnt, W2, b2.reshape(1, 256), sc2.reshape(1, 256),
      sh2.reshape(1, 256), Wout, bout.reshape(1, 128))


def _tc_replicate(x):
    def body(x_ref, o_ref):
        row = jnp.concatenate(
            [x_ref[...], jnp.ones((BN_TC, 1), jnp.float32),
             jnp.zeros((BN_TC, 15), jnp.float32)], axis=1)
        for r in range(8):
            o_ref[r] = row

    return pl.pallas_call(
        body, grid=(N_NODES // BN_TC,),
        in_specs=[pl.BlockSpec((BN_TC, 128), lambda i: (i, 0))],
        out_specs=pl.BlockSpec((8, BN_TC, 144), lambda i: (0, i, 0)),
        out_shape=jax.ShapeDtypeStruct((8, N_NODES, 144), jnp.float32),
    )(x)


def _tc_edge_prep(src, dst, N, e_per, pad_per):
    W = e_per + pad_per

    def body(s_ref, d_ref, so_ref, do_ref):
        so_ref[:, :e_per] = s_ref[...]
        do_ref[:, :e_per] = d_ref[...]
        so_ref[:, e_per:] = jnp.zeros((8, pad_per), jnp.int32)
        do_ref[:, e_per:] = N + lax.broadcasted_iota(jnp.int32, (8, pad_per), 1)

    so, do = pl.pallas_call(
        body, grid=(NW // 8,),
        in_specs=[pl.BlockSpec((8, e_per), lambda i: (i, 0)),
                  pl.BlockSpec((8, e_per), lambda i: (i, 0))],
        out_specs=[pl.BlockSpec((8, W), lambda i: (i, 0)),
                   pl.BlockSpec((8, W), lambda i: (i, 0))],
        out_shape=[jax.ShapeDtypeStruct((NW, W), jnp.int32),
                   jax.ShapeDtypeStruct((NW, W), jnp.int32)],
    )(src.reshape(NW, e_per), dst.reshape(NW, e_per))
    return so.reshape(NW, NC, CHUNK), do.reshape(NW, NC, CHUNK)


def kernel(x, edge_index, batch, W1, b1, g1, be1, rm1, rv1, Wg, bg, gg, beg,
           rmg, rvg, W2, b2, g2, be2, rm2, rv2, Wout, bout):
    del batch
    N = x.shape[0]
    E = edge_index.shape[1]
    src = edge_index[0]
    dst = edge_index[1]

    e_per = E // NW
    pad_per = NC * CHUNK - e_per
    src_r, dst_r = _tc_edge_prep(src, dst, N, e_per, pad_per)

    eps = 1e-5
    sc1 = g1 / jnp.sqrt(rv1 + eps)
    sh1 = be1 - rm1 * sc1
    scg = gg / jnp.sqrt(rvg + eps)
    shg = beg - rmg * scg
    sc2 = g2 / jnp.sqrt(rv2 + eps)
    sh2 = be2 - rm2 * sc2

    x_aug = _tc_replicate(x)

    S1p = _make_sc_agg(144)(x_aug, src_r, dst_r)
    h1, cnt = _tc_layer(S1p, None, W1, b1, sc1, sh1, first=True)

    S2p = _make_sc_agg(128)(h1, src_r, dst_r)
    h2 = _tc_layer(S2p, cnt, Wg, bg, scg, shg, first=False)

    S3p = _make_sc_agg(128)(h2, src_r, dst_r)
    out = _tc_final(S3p, cnt, W2, b2, sc2, sh2, Wout, bout)

    l1_reg = jnp.array(0.0, dtype=jnp.float32)
    return (out, l1_reg)


# speedup vs baseline: 1.2618x; 1.2618x over previous
import functools

import jax
import jax.numpy as jnp
from jax import lax
from jax.experimental import pallas as pl
from jax.experimental.pallas import tpu as pltpu
from jax.experimental.pallas import tpu_sc as plsc

N_NODES = 10000
N_ACC = 10240
NW = 32
CHUNK = 128
NC = 80
E_PAD = NW * NC * CHUNK
NB = 2
BN_TC = 1000


def _make_sc_agg(D, with_counts=False):
    mesh = plsc.VectorSubcoreMesh(core_axis_name="c", subcore_axis_name="s")
    NI = 2 * NB

    out_type = [jax.ShapeDtypeStruct((2, N_NODES, D), jnp.float32)]
    scratch = [
        pltpu.VMEM((NI, CHUNK), jnp.int32),
        pltpu.VMEM((NI, CHUNK), jnp.int32),
        pltpu.VMEM((NB, CHUNK, D), jnp.float32),
        pltpu.VMEM_SHARED((N_ACC, D), jnp.float32),
    ]
    if with_counts:
        out_type.append(jax.ShapeDtypeStruct((2, N_NODES, 16), jnp.float32))
        scratch += [
            pltpu.VMEM((CHUNK, 16), jnp.float32),
            pltpu.VMEM((CHUNK, 16), jnp.float32),
            pltpu.VMEM_SHARED((N_ACC, 16), jnp.float32),
        ]

    @functools.partial(
        pl.kernel,
        out_type=out_type,
        mesh=mesh,
        scratch_types=scratch + [pltpu.SemaphoreType.DMA] * (NI + NI + NB),
        compiler_params=pltpu.CompilerParams(use_tc_tiling_on_sc=False),
    )
    def k(table, src_r, dst_r, *rest):
        if with_counts:
            (out, outc, sidx, didx, rows, acc, cones, cbuf, cacc) = rest[:9]
            sems = rest[9:]
        else:
            (out, sidx, didx, rows, acc) = rest[:5]
            sems = rest[5:]
        sisem = sems[:NI]
        disem = sems[NI:2 * NI]
        gsem = sems[2 * NI:]
        c = lax.axis_index("c")
        s = lax.axis_index("s")
        wid = c * 16 + s
        tab = table.at[c * 4 + s // 4]

        zvec = jnp.zeros((16,), jnp.float32)

        def zrow(i, _):
            for j in range(D // 16):
                rows[0, i, pl.ds(j * 16, 16)] = zvec
            return 0

        lax.fori_loop(0, CHUNK, zrow, 0)
        if with_counts:
            ovec = jnp.ones((16,), jnp.float32)

            def crow(i, _):
                cbuf[i, pl.ds(0, 16)] = zvec
                cones[i, pl.ds(0, 16)] = ovec
                return 0

            lax.fori_loop(0, CHUNK, crow, 0)
        for z in range(5):
            pltpu.sync_copy(rows.at[0], acc.at[pl.ds(s * 640 + z * CHUNK, CHUNK)])
            if with_counts:
                pltpu.sync_copy(
                    cbuf, cacc.at[pl.ds(s * 640 + z * CHUNK, CHUNK)])
        plsc.subcore_barrier()

        def fire_idx(j, sl):
            pltpu.async_copy(src_r.at[wid, j], sidx.at[sl], sisem[sl])
            pltpu.async_copy(dst_r.at[wid, j], didx.at[sl], disem[sl])

        def fire_gather(j, sl, rsl):
            pltpu.make_async_copy(
                src_r.at[0, 0], sidx.at[sl], sisem[sl]).wait()
            pltpu.async_copy(tab.at[sidx.at[sl]], rows.at[rsl], gsem[rsl])

        def do_scatter(j, sl, rsl):
            pltpu.make_async_copy(
                tab.at[sidx.at[0]], rows.at[rsl], gsem[rsl]).wait()
            pltpu.make_async_copy(
                dst_r.at[0, 0], didx.at[sl], disem[sl]).wait()
            pltpu.sync_copy(rows.at[rsl], acc.at[didx.at[sl]], add=True)
            if with_counts:
                pltpu.sync_copy(cones, cacc.at[didx.at[sl]], add=True)

        for i in range(2 * NB):
            if i >= NB:
                fire_gather(i - NB, (i - NB) % NI, (i - NB) % NB)
            fire_idx(i, i % NI)

        def body(g, _):
            for u in range(2 * NB):
                i = 2 * NB + g * 2 * NB + u
                do_scatter(i - 2 * NB, u, u % NB)
                fire_gather(i - NB, (u + NB) % NI, u % NB)
                fire_idx(i, u)
            return 0

        lax.fori_loop(0, (NC - 2 * NB) // (2 * NB), body, 0)

        for i in range(NC, NC + 2 * NB):
            do_scatter(i - 2 * NB, (i - 2 * NB) % NI, (i - 2 * NB) % NB)
            if i - NB < NC:
                fire_gather(i - NB, (i - NB) % NI, (i - NB) % NB)
        plsc.subcore_barrier()

        for z in range(5):
            r0 = s * 625 + z * 125
            pltpu.sync_copy(acc.at[pl.ds(r0, 125)], rows.at[0, pl.ds(0, 125)])
            pltpu.sync_copy(rows.at[0, pl.ds(0, 125)], out.at[c, pl.ds(r0, 125)])
            if with_counts:
                pltpu.sync_copy(cacc.at[pl.ds(r0, 125)], cbuf.at[pl.ds(0, 125)])
                pltpu.sync_copy(
                    cbuf.at[pl.ds(0, 125)], outc.at[c, pl.ds(r0, 125)])

    if with_counts:
        return k
    return lambda *a: k(*a)[0]


def _tc_layer(Sp, cnt, W, b, sc, sh, first):
    D = Sp.shape[-1]
    H = W.shape[0]
    grid = (N_NODES // BN_TC,)

    def body(*refs):
        if first:
            p_ref, c_in_ref, w_ref, b_ref, sc_ref, sh_ref, h_ref, c_ref = refs
        else:
            p_ref, c_in_ref, w_ref, b_ref, sc_ref, sh_ref, h_ref = refs
        S = p_ref[0] + p_ref[1]
        if first:
            cc = (c_in_ref[0] + c_in_ref[1])[:, 0:1]
        else:
            cc = c_in_ref[:, 0:1]
        r = jnp.where(cc > 0, 1.0 / jnp.maximum(cc, 1.0), 0.0)
        A = S * r
        Z = lax.dot_general(A, w_ref[...], (((1,), (1,)), ((), ())),
                            preferred_element_type=jnp.float32)
        Z = jnp.where(cc > 0, Z + b_ref[...], 0.0)
        Z = Z * sc_ref[...] + sh_ref[...]
        h = jnp.where(Z > 0, Z, jnp.exp(Z) - 1.0)
        for _r in range(8):
            h_ref[_r] = h
        if first:
            c_ref[...] = jnp.broadcast_to(cc, (BN_TC, 8))

    in_specs = [pl.BlockSpec((2, BN_TC, D), lambda i: (0, i, 0))]
    if first:
        in_specs.append(pl.BlockSpec((2, BN_TC, 16), lambda i: (0, i, 0)))
    else:
        in_specs.append(pl.BlockSpec((BN_TC, 8), lambda i: (i, 0)))
    in_specs += [
        pl.BlockSpec(W.shape, lambda i: (0, 0)),
        pl.BlockSpec((1, H), lambda i: (0, 0)),
        pl.BlockSpec((1, H), lambda i: (0, 0)),
        pl.BlockSpec((1, H), lambda i: (0, 0)),
    ]
    out_shape = [jax.ShapeDtypeStruct((8, N_NODES, H), jnp.float32)]
    out_specs = [pl.BlockSpec((8, BN_TC, H), lambda i: (0, i, 0))]
    if first:
        out_shape.append(jax.ShapeDtypeStruct((N_NODES, 8), jnp.float32))
        out_specs.append(pl.BlockSpec((BN_TC, 8), lambda i: (i, 0)))

    args = [Sp, cnt]
    args += [W, b.reshape(1, H), sc.reshape(1, H), sh.reshape(1, H)]
    res = pl.pallas_call(
        body, grid=grid, in_specs=in_specs, out_specs=out_specs,
        out_shape=out_shape)(*args)
    return res if first else res[0]


def _tc_final(Sp, cnt, W2, b2, sc2, sh2, Wout, bout):
    D = Sp.shape[-1]
    grid = (N_NODES // BN_TC,)

    def body(p_ref, c_ref, w2_ref, b2_ref, sc_ref, sh_ref, wo_ref, bo_ref,
             o_ref):
        P = p_ref[0] + p_ref[1]
        cc = c_ref[:, 0:1]
        r = jnp.where(cc > 0, 1.0 / jnp.maximum(cc, 1.0), 0.0)
        A = P * r
        Z = lax.dot_general(A, w2_ref[...], (((1,), (1,)), ((), ())),
                            preferred_element_type=jnp.float32)
        Z = jnp.where(cc > 0, Z + b2_ref[...], 0.0)
        Z = Z * sc_ref[...] + sh_ref[...]
        h3 = jnp.where(Z > 0, Z, jnp.exp(Z) - 1.0)
        o_ref[...] = lax.dot_general(h3, wo_ref[...], (((1,), (1,)), ((), ())),
                                     preferred_element_type=jnp.float32) + bo_ref[...]

    return pl.pallas_call(
        body, grid=grid,
        in_specs=[
            pl.BlockSpec((2, BN_TC, D), lambda i: (0, i, 0)),
            pl.BlockSpec((BN_TC, 8), lambda i: (i, 0)),
            pl.BlockSpec(W2.shape, lambda i: (0, 0)),
            pl.BlockSpec((1, 256), lambda i: (0, 0)),
            pl.BlockSpec((1, 256), lambda i: (0, 0)),
            pl.BlockSpec((1, 256), lambda i: (0, 0)),
            pl.BlockSpec(Wout.shape, lambda i: (0, 0)),
            pl.BlockSpec((1, 128), lambda i: (0, 0)),
        ],
        out_specs=pl.BlockSpec((BN_TC, 128), lambda i: (i, 0)),
        out_shape=jax.ShapeDtypeStruct((N_NODES, 128), jnp.float32),
    )(Sp, cnt, W2, b2.reshape(1, 256), sc2.reshape(1, 256),
      sh2.reshape(1, 256), Wout, bout.reshape(1, 128))


def _tc_edge_prep(src, dst, N, e_per, pad_per):
    W = e_per + pad_per

    def body(s_ref, d_ref, so_ref, do_ref):
        so_ref[:, :e_per] = s_ref[...]
        do_ref[:, :e_per] = d_ref[...]
        so_ref[:, e_per:] = jnp.zeros((8, pad_per), jnp.int32)
        do_ref[:, e_per:] = N + lax.broadcasted_iota(jnp.int32, (8, pad_per), 1)

    so, do = pl.pallas_call(
        body, grid=(NW // 8,),
        in_specs=[pl.BlockSpec((8, e_per), lambda i: (i, 0)),
                  pl.BlockSpec((8, e_per), lambda i: (i, 0))],
        out_specs=[pl.BlockSpec((8, W), lambda i: (i, 0)),
                   pl.BlockSpec((8, W), lambda i: (i, 0))],
        out_shape=[jax.ShapeDtypeStruct((NW, W), jnp.int32),
                   jax.ShapeDtypeStruct((NW, W), jnp.int32)],
    )(src.reshape(NW, e_per), dst.reshape(NW, e_per))
    return so.reshape(NW, NC, CHUNK), do.reshape(NW, NC, CHUNK)


def kernel(x, edge_index, batch, W1, b1, g1, be1, rm1, rv1, Wg, bg, gg, beg,
           rmg, rvg, W2, b2, g2, be2, rm2, rv2, Wout, bout):
    del batch
    N = x.shape[0]
    E = edge_index.shape[1]
    src = edge_index[0]
    dst = edge_index[1]

    e_per = E // NW
    pad_per = NC * CHUNK - e_per
    src_r, dst_r = _tc_edge_prep(src, dst, N, e_per, pad_per)

    eps = 1e-5
    sc1 = g1 / jnp.sqrt(rv1 + eps)
    sh1 = be1 - rm1 * sc1
    scg = gg / jnp.sqrt(rvg + eps)
    shg = beg - rmg * scg
    sc2 = g2 / jnp.sqrt(rv2 + eps)
    sh2 = be2 - rm2 * sc2

    x_rep = jnp.broadcast_to(x[None], (8, N, 128)) + jnp.zeros(
        (8, 1, 1), jnp.float32)

    S1p, C1p = _make_sc_agg(128, with_counts=True)(x_rep, src_r, dst_r)
    h1, cnt = _tc_layer(S1p, C1p, W1, b1, sc1, sh1, first=True)

    S2p = _make_sc_agg(128)(h1, src_r, dst_r)
    h2 = _tc_layer(S2p, cnt, Wg, bg, scg, shg, first=False)

    S3p = _make_sc_agg(128)(h2, src_r, dst_r)
    out = _tc_final(S3p, cnt, W2, b2, sc2, sh2, Wout, bout)

    l1_reg = jnp.array(0.0, dtype=jnp.float32)
    return (out, l1_reg)


# speedup vs baseline: 1.2863x; 1.0194x over previous
import functools

import jax
import jax.numpy as jnp
from jax import lax
from jax.experimental import pallas as pl
from jax.experimental.pallas import tpu as pltpu
from jax.experimental.pallas import tpu_sc as plsc

N_NODES = 10000
N_ACC = 10240
NW = 32
CHUNK = 128
NC = 80
E_PAD = NW * NC * CHUNK
NB = 2
BN_TC = 1000


def _make_sc_agg(D, with_counts=False):
    mesh = plsc.VectorSubcoreMesh(core_axis_name="c", subcore_axis_name="s")
    NI = 2 * NB

    out_type = [jax.ShapeDtypeStruct((2, N_NODES, D), jnp.float32)]
    scratch = [
        pltpu.VMEM((NI, CHUNK), jnp.int32),
        pltpu.VMEM((NI, CHUNK), jnp.int32),
        pltpu.VMEM((NB, CHUNK, D), jnp.float32),
        pltpu.VMEM_SHARED((N_ACC, D), jnp.float32),
    ]
    if with_counts:
        out_type.append(jax.ShapeDtypeStruct((2, N_NODES, 16), jnp.float32))
        scratch += [
            pltpu.VMEM((CHUNK, 16), jnp.float32),
            pltpu.VMEM((CHUNK, 16), jnp.float32),
            pltpu.VMEM_SHARED((N_ACC, 16), jnp.float32),
        ]

    @functools.partial(
        pl.kernel,
        out_type=out_type,
        mesh=mesh,
        scratch_types=scratch + [pltpu.SemaphoreType.DMA] * (NI + NI + NB),
        compiler_params=pltpu.CompilerParams(use_tc_tiling_on_sc=False),
    )
    def k(table, src_r, dst_r, *rest):
        if with_counts:
            (out, outc, sidx, didx, rows, acc, cones, cbuf, cacc) = rest[:9]
            sems = rest[9:]
        else:
            (out, sidx, didx, rows, acc) = rest[:5]
            sems = rest[5:]
        sisem = sems[:NI]
        disem = sems[NI:2 * NI]
        gsem = sems[2 * NI:]
        c = lax.axis_index("c")
        s = lax.axis_index("s")
        wid = c * 16 + s
        tab = table.at[c * 4 + s // 4]

        zvec = jnp.zeros((16,), jnp.float32)

        def zrow(i, _):
            for j in range(D // 16):
                rows[0, i, pl.ds(j * 16, 16)] = zvec
            return 0

        lax.fori_loop(0, CHUNK, zrow, 0)
        if with_counts:
            ovec = jnp.ones((16,), jnp.float32)

            def crow(i, _):
                cbuf[i, pl.ds(0, 16)] = zvec
                cones[i, pl.ds(0, 16)] = ovec
                return 0

            lax.fori_loop(0, CHUNK, crow, 0)
        for z in range(5):
            pltpu.sync_copy(rows.at[0], acc.at[pl.ds(s * 640 + z * CHUNK, CHUNK)])
            if with_counts:
                pltpu.sync_copy(
                    cbuf, cacc.at[pl.ds(s * 640 + z * CHUNK, CHUNK)])
        plsc.subcore_barrier()

        def fire_idx(j, sl):
            pltpu.async_copy(src_r.at[wid, j], sidx.at[sl], sisem[sl])
            pltpu.async_copy(dst_r.at[wid, j], didx.at[sl], disem[sl])

        def fire_gather(j, sl, rsl):
            pltpu.make_async_copy(
                src_r.at[0, 0], sidx.at[sl], sisem[sl]).wait()
            pltpu.async_copy(tab.at[sidx.at[sl]], rows.at[rsl], gsem[rsl])

        def do_scatter(j, sl, rsl):
            pltpu.make_async_copy(
                tab.at[sidx.at[0]], rows.at[rsl], gsem[rsl]).wait()
            pltpu.make_async_copy(
                dst_r.at[0, 0], didx.at[sl], disem[sl]).wait()
            pltpu.sync_copy(rows.at[rsl], acc.at[didx.at[sl]], add=True)
            if with_counts:
                pltpu.sync_copy(cones, cacc.at[didx.at[sl]], add=True)

        for i in range(2 * NB):
            if i >= NB:
                fire_gather(i - NB, (i - NB) % NI, (i - NB) % NB)
            fire_idx(i, i % NI)

        def body(g, _):
            for u in range(2 * NB):
                i = 2 * NB + g * 2 * NB + u
                do_scatter(i - 2 * NB, u, u % NB)
                fire_gather(i - NB, (u + NB) % NI, u % NB)
                fire_idx(i, u)
            return 0

        lax.fori_loop(0, (NC - 2 * NB) // (2 * NB), body, 0)

        for i in range(NC, NC + 2 * NB):
            do_scatter(i - 2 * NB, (i - 2 * NB) % NI, (i - 2 * NB) % NB)
            if i - NB < NC:
                fire_gather(i - NB, (i - NB) % NI, (i - NB) % NB)
        plsc.subcore_barrier()

        for z in range(5):
            r0 = s * 625 + z * 125
            pltpu.sync_copy(acc.at[pl.ds(r0, 125)], rows.at[0, pl.ds(0, 125)])
            pltpu.sync_copy(rows.at[0, pl.ds(0, 125)], out.at[c, pl.ds(r0, 125)])
            if with_counts:
                pltpu.sync_copy(cacc.at[pl.ds(r0, 125)], cbuf.at[pl.ds(0, 125)])
                pltpu.sync_copy(
                    cbuf.at[pl.ds(0, 125)], outc.at[c, pl.ds(r0, 125)])

    if with_counts:
        return k
    return lambda *a: k(*a)[0]


def _tc_layer(Sp, cnt, W, b, sc, sh, first):
    D = Sp.shape[-1]
    H = W.shape[0]
    grid = (N_NODES // BN_TC,)

    def body(*refs):
        if first:
            p_ref, c_in_ref, w_ref, b_ref, sc_ref, sh_ref, h_ref, c_ref = refs
        else:
            p_ref, c_in_ref, w_ref, b_ref, sc_ref, sh_ref, h_ref = refs
        S = p_ref[0] + p_ref[1]
        if first:
            cc = (c_in_ref[0] + c_in_ref[1])[:, 0:1]
        else:
            cc = c_in_ref[:, 0:1]
        r = jnp.where(cc > 0, 1.0 / jnp.maximum(cc, 1.0), 0.0)
        A = S * r
        Z = lax.dot_general(A, w_ref[...], (((1,), (1,)), ((), ())),
                            preferred_element_type=jnp.float32)
        Z = jnp.where(cc > 0, Z + b_ref[...], 0.0)
        Z = Z * sc_ref[...] + sh_ref[...]
        h = jnp.where(Z > 0, Z, jnp.exp(Z) - 1.0)
        for _r in range(8):
            h_ref[_r] = h
        if first:
            c_ref[...] = jnp.broadcast_to(cc, (BN_TC, 8))

    in_specs = [pl.BlockSpec((2, BN_TC, D), lambda i: (0, i, 0))]
    if first:
        in_specs.append(pl.BlockSpec((2, BN_TC, 16), lambda i: (0, i, 0)))
    else:
        in_specs.append(pl.BlockSpec((BN_TC, 8), lambda i: (i, 0)))
    in_specs += [
        pl.BlockSpec(W.shape, lambda i: (0, 0)),
        pl.BlockSpec((1, H), lambda i: (0, 0)),
        pl.BlockSpec((1, H), lambda i: (0, 0)),
        pl.BlockSpec((1, H), lambda i: (0, 0)),
    ]
    out_shape = [jax.ShapeDtypeStruct((8, N_NODES, H), jnp.float32)]
    out_specs = [pl.BlockSpec((8, BN_TC, H), lambda i: (0, i, 0))]
    if first:
        out_shape.append(jax.ShapeDtypeStruct((N_NODES, 8), jnp.float32))
        out_specs.append(pl.BlockSpec((BN_TC, 8), lambda i: (i, 0)))

    args = [Sp, cnt]
    args += [W, b.reshape(1, H), sc.reshape(1, H), sh.reshape(1, H)]
    res = pl.pallas_call(
        body, grid=grid, in_specs=in_specs, out_specs=out_specs,
        out_shape=out_shape)(*args)
    return res if first else res[0]


def _tc_final(Sp, cnt, W2, b2, sc2, sh2, Wout, bout):
    D = Sp.shape[-1]
    grid = (N_NODES // BN_TC,)

    def body(p_ref, c_ref, w2_ref, b2_ref, sc_ref, sh_ref, wo_ref, bo_ref,
             o_ref):
        P = p_ref[0] + p_ref[1]
        cc = c_ref[:, 0:1]
        r = jnp.where(cc > 0, 1.0 / jnp.maximum(cc, 1.0), 0.0)
        A = P * r
        Z = lax.dot_general(A, w2_ref[...], (((1,), (1,)), ((), ())),
                            preferred_element_type=jnp.float32)
        Z = jnp.where(cc > 0, Z + b2_ref[...], 0.0)
        Z = Z * sc_ref[...] + sh_ref[...]
        h3 = jnp.where(Z > 0, Z, jnp.exp(Z) - 1.0)
        o_ref[...] = lax.dot_general(h3, wo_ref[...], (((1,), (1,)), ((), ())),
                                     preferred_element_type=jnp.float32) + bo_ref[...]

    return pl.pallas_call(
        body, grid=grid,
        in_specs=[
            pl.BlockSpec((2, BN_TC, D), lambda i: (0, i, 0)),
            pl.BlockSpec((BN_TC, 8), lambda i: (i, 0)),
            pl.BlockSpec(W2.shape, lambda i: (0, 0)),
            pl.BlockSpec((1, 256), lambda i: (0, 0)),
            pl.BlockSpec((1, 256), lambda i: (0, 0)),
            pl.BlockSpec((1, 256), lambda i: (0, 0)),
            pl.BlockSpec(Wout.shape, lambda i: (0, 0)),
            pl.BlockSpec((1, 128), lambda i: (0, 0)),
        ],
        out_specs=pl.BlockSpec((BN_TC, 128), lambda i: (i, 0)),
        out_shape=jax.ShapeDtypeStruct((N_NODES, 128), jnp.float32),
    )(Sp, cnt, W2, b2.reshape(1, 256), sc2.reshape(1, 256),
      sh2.reshape(1, 256), Wout, bout.reshape(1, 128))


def _tc_edge_prep(src, N, e_per, pad_per):
    W = e_per + pad_per

    def body(e_ref, so_ref, do_ref):
        so_ref[:, :e_per] = e_ref[0]
        do_ref[:, :e_per] = e_ref[1]
        so_ref[:, e_per:] = jnp.zeros((8, pad_per), jnp.int32)
        do_ref[:, e_per:] = N + lax.broadcasted_iota(jnp.int32, (8, pad_per), 1)

    so, do = pl.pallas_call(
        body, grid=(NW // 8,),
        in_specs=[pl.BlockSpec((2, 8, e_per), lambda i: (0, i, 0))],
        out_specs=[pl.BlockSpec((8, W), lambda i: (i, 0)),
                   pl.BlockSpec((8, W), lambda i: (i, 0))],
        out_shape=[jax.ShapeDtypeStruct((NW, W), jnp.int32),
                   jax.ShapeDtypeStruct((NW, W), jnp.int32)],
    )(src.reshape(2, NW, e_per))
    return so.reshape(NW, NC, CHUNK), do.reshape(NW, NC, CHUNK)


def kernel(x, edge_index, batch, W1, b1, g1, be1, rm1, rv1, Wg, bg, gg, beg,
           rmg, rvg, W2, b2, g2, be2, rm2, rv2, Wout, bout):
    del batch
    N = x.shape[0]
    E = edge_index.shape[1]
    src = edge_index[0]
    dst = edge_index[1]

    e_per = E // NW
    pad_per = NC * CHUNK - e_per
    src_r, dst_r = _tc_edge_prep(edge_index, N, e_per, pad_per)

    eps = 1e-5
    sc1 = g1 / jnp.sqrt(rv1 + eps)
    sh1 = be1 - rm1 * sc1
    scg = gg / jnp.sqrt(rvg + eps)
    shg = beg - rmg * scg
    sc2 = g2 / jnp.sqrt(rv2 + eps)
    sh2 = be2 - rm2 * sc2

    x_rep = jnp.broadcast_to(x[None], (8, N, 128)) + jnp.zeros(
        (8, 1, 1), jnp.float32)

    S1p, C1p = _make_sc_agg(128, with_counts=True)(x_rep, src_r, dst_r)
    h1, cnt = _tc_layer(S1p, C1p, W1, b1, sc1, sh1, first=True)

    S2p = _make_sc_agg(128)(h1, src_r, dst_r)
    h2 = _tc_layer(S2p, cnt, Wg, bg, scg, shg, first=False)

    S3p = _make_sc_agg(128)(h2, src_r, dst_r)
    out = _tc_final(S3p, cnt, W2, b2, sc2, sh2, Wout, bout)

    l1_reg = jnp.array(0.0, dtype=jnp.float32)
    return (out, l1_reg)


# speedup vs baseline: 1.4941x; 1.1616x over previous
import functools

import jax
import jax.numpy as jnp
from jax import lax
from jax.experimental import pallas as pl
from jax.experimental.pallas import tpu as pltpu
from jax.experimental.pallas import tpu_sc as plsc

N_NODES = 10000
N_ACC = 10240
NW = 32
CHUNK = 128
NC = 80
E_PAD = NW * NC * CHUNK
NB = 2
BN_TC = 1000


def _make_sc_agg(D, with_counts=False, CHUNK=128, NC=80, NB=2):
    mesh = plsc.VectorSubcoreMesh(core_axis_name="c", subcore_axis_name="s")
    NI = 2 * NB

    out_type = [jax.ShapeDtypeStruct((2, N_NODES, D), jnp.float32)]
    scratch = [
        pltpu.VMEM((NI, CHUNK), jnp.int32),
        pltpu.VMEM((NI, CHUNK), jnp.int32),
        pltpu.VMEM((NB, CHUNK, D), jnp.float32),
        pltpu.VMEM_SHARED((N_ACC, D), jnp.float32),
    ]
    if with_counts:
        out_type.append(jax.ShapeDtypeStruct((2, N_NODES, 16), jnp.float32))
        scratch += [
            pltpu.VMEM((CHUNK, 16), jnp.float32),
            pltpu.VMEM((CHUNK, 16), jnp.float32),
            pltpu.VMEM_SHARED((N_ACC, 16), jnp.float32),
        ]

    @functools.partial(
        pl.kernel,
        out_type=out_type,
        mesh=mesh,
        scratch_types=scratch + [pltpu.SemaphoreType.DMA] * (NI + NI + NB),
        compiler_params=pltpu.CompilerParams(use_tc_tiling_on_sc=False),
    )
    def k(table, src_r, dst_r, *rest):
        if with_counts:
            (out, outc, sidx, didx, rows, acc, cones, cbuf, cacc) = rest[:9]
            sems = rest[9:]
        else:
            (out, sidx, didx, rows, acc) = rest[:5]
            sems = rest[5:]
        sisem = sems[:NI]
        disem = sems[NI:2 * NI]
        gsem = sems[2 * NI:]
        c = lax.axis_index("c")
        s = lax.axis_index("s")
        wid = c * 16 + s
        tab = table.at[c * 4 + s // 4]

        zvec = jnp.zeros((16,), jnp.float32)

        def zrow(i, _):
            for j in range(D // 16):
                rows[0, i, pl.ds(j * 16, 16)] = zvec
            return 0

        lax.fori_loop(0, CHUNK, zrow, 0)
        if with_counts:
            ovec = jnp.ones((16,), jnp.float32)

            def crow(i, _):
                cbuf[i, pl.ds(0, 16)] = zvec
                cones[i, pl.ds(0, 16)] = ovec
                return 0

            lax.fori_loop(0, CHUNK, crow, 0)
        ZC = 128 if CHUNK >= 128 else 80
        for z in range(640 // ZC):
            pltpu.sync_copy(rows.at[0, pl.ds(0, ZC)],
                            acc.at[pl.ds(s * 640 + z * ZC, ZC)])
            if with_counts:
                pltpu.sync_copy(
                    cbuf, cacc.at[pl.ds(s * 640 + z * CHUNK, CHUNK)])
        plsc.subcore_barrier()

        def fire_idx(j, sl):
            pltpu.async_copy(src_r.at[wid, j], sidx.at[sl], sisem[sl])
            pltpu.async_copy(dst_r.at[wid, j], didx.at[sl], disem[sl])

        def fire_gather(j, sl, rsl):
            pltpu.make_async_copy(
                src_r.at[0, 0], sidx.at[sl], sisem[sl]).wait()
            pltpu.async_copy(tab.at[sidx.at[sl]], rows.at[rsl], gsem[rsl])

        def do_scatter(j, sl, rsl):
            pltpu.make_async_copy(
                tab.at[sidx.at[0]], rows.at[rsl], gsem[rsl]).wait()
            pltpu.make_async_copy(
                dst_r.at[0, 0], didx.at[sl], disem[sl]).wait()
            pltpu.sync_copy(rows.at[rsl], acc.at[didx.at[sl]], add=True)
            if with_counts:
                pltpu.sync_copy(cones, cacc.at[didx.at[sl]], add=True)

        for i in range(2 * NB):
            if i >= NB:
                fire_gather(i - NB, (i - NB) % NI, (i - NB) % NB)
            fire_idx(i, i % NI)

        def body(g, _):
            for u in range(2 * NB):
                i = 2 * NB + g * 2 * NB + u
                do_scatter(i - 2 * NB, u, u % NB)
                fire_gather(i - NB, (u + NB) % NI, u % NB)
                fire_idx(i, u)
            return 0

        lax.fori_loop(0, (NC - 2 * NB) // (2 * NB), body, 0)

        for i in range(NC, NC + 2 * NB):
            do_scatter(i - 2 * NB, (i - 2 * NB) % NI, (i - 2 * NB) % NB)
            if i - NB < NC:
                fire_gather(i - NB, (i - NB) % NI, (i - NB) % NB)
        plsc.subcore_barrier()

        sizes = []
        rem = 625
        while rem:
            t = min(rem, CHUNK if CHUNK < 125 else 125)
            sizes.append(t)
            rem -= t
        r0 = 0
        for t in sizes:
            ro = s * 625 + r0
            pltpu.sync_copy(acc.at[pl.ds(ro, t)], rows.at[0, pl.ds(0, t)])
            pltpu.sync_copy(rows.at[0, pl.ds(0, t)], out.at[c, pl.ds(ro, t)])
            if with_counts:
                pltpu.sync_copy(cacc.at[pl.ds(ro, t)], cbuf.at[pl.ds(0, t)])
                pltpu.sync_copy(
                    cbuf.at[pl.ds(0, t)], outc.at[c, pl.ds(ro, t)])
            r0 += t

    if with_counts:
        return k
    return lambda *a: k(*a)[0]


def _tc_layer(Sp, cnt, W, b, sc, sh, first):
    D = Sp.shape[-1]
    H = W.shape[0]
    grid = (N_NODES // BN_TC,)

    def body(*refs):
        if first:
            p_ref, c_in_ref, w_ref, b_ref, sc_ref, sh_ref, h_ref, c_ref = refs
        else:
            p_ref, c_in_ref, w_ref, b_ref, sc_ref, sh_ref, h_ref = refs
        S = p_ref[0] + p_ref[1]
        if first:
            cc = (c_in_ref[0] + c_in_ref[1])[:, 0:1]
        else:
            cc = c_in_ref[:, 0:1]
        r = jnp.where(cc > 0, 1.0 / jnp.maximum(cc, 1.0), 0.0)
        A = S * r
        Z = lax.dot_general(A, w_ref[...], (((1,), (1,)), ((), ())),
                            preferred_element_type=jnp.float32)
        Z = jnp.where(cc > 0, Z + b_ref[...], 0.0)
        Z = Z * sc_ref[...] + sh_ref[...]
        h = jnp.where(Z > 0, Z, jnp.exp(Z) - 1.0)
        for _r in range(8):
            h_ref[_r] = h
        if first:
            c_ref[...] = jnp.broadcast_to(cc, (BN_TC, 8))

    in_specs = [pl.BlockSpec((2, BN_TC, D), lambda i: (0, i, 0))]
    if first:
        in_specs.append(pl.BlockSpec((2, BN_TC, 16), lambda i: (0, i, 0)))
    else:
        in_specs.append(pl.BlockSpec((BN_TC, 8), lambda i: (i, 0)))
    in_specs += [
        pl.BlockSpec(W.shape, lambda i: (0, 0)),
        pl.BlockSpec((1, H), lambda i: (0, 0)),
        pl.BlockSpec((1, H), lambda i: (0, 0)),
        pl.BlockSpec((1, H), lambda i: (0, 0)),
    ]
    out_shape = [jax.ShapeDtypeStruct((8, N_NODES, H), jnp.float32)]
    out_specs = [pl.BlockSpec((8, BN_TC, H), lambda i: (0, i, 0))]
    if first:
        out_shape.append(jax.ShapeDtypeStruct((N_NODES, 8), jnp.float32))
        out_specs.append(pl.BlockSpec((BN_TC, 8), lambda i: (i, 0)))

    args = [Sp, cnt]
    args += [W, b.reshape(1, H), sc.reshape(1, H), sh.reshape(1, H)]
    res = pl.pallas_call(
        body, grid=grid, in_specs=in_specs, out_specs=out_specs,
        out_shape=out_shape)(*args)
    return res if first else res[0]


def _tc_final(Sp, cnt, W2, b2, sc2, sh2, Wout, bout):
    D = Sp.shape[-1]
    grid = (N_NODES // BN_TC,)

    def body(p_ref, c_ref, w2_ref, b2_ref, sc_ref, sh_ref, wo_ref, bo_ref,
             o_ref):
        P = p_ref[0] + p_ref[1]
        cc = c_ref[:, 0:1]
        r = jnp.where(cc > 0, 1.0 / jnp.maximum(cc, 1.0), 0.0)
        A = P * r
        Z = lax.dot_general(A, w2_ref[...], (((1,), (1,)), ((), ())),
                            preferred_element_type=jnp.float32)
        Z = jnp.where(cc > 0, Z + b2_ref[...], 0.0)
        Z = Z * sc_ref[...] + sh_ref[...]
        h3 = jnp.where(Z > 0, Z, jnp.exp(Z) - 1.0)
        o_ref[...] = lax.dot_general(h3, wo_ref[...], (((1,), (1,)), ((), ())),
                                     preferred_element_type=jnp.float32) + bo_ref[...]

    return pl.pallas_call(
        body, grid=grid,
        in_specs=[
            pl.BlockSpec((2, BN_TC, D), lambda i: (0, i, 0)),
            pl.BlockSpec((BN_TC, 8), lambda i: (i, 0)),
            pl.BlockSpec(W2.shape, lambda i: (0, 0)),
            pl.BlockSpec((1, 256), lambda i: (0, 0)),
            pl.BlockSpec((1, 256), lambda i: (0, 0)),
            pl.BlockSpec((1, 256), lambda i: (0, 0)),
            pl.BlockSpec(Wout.shape, lambda i: (0, 0)),
            pl.BlockSpec((1, 128), lambda i: (0, 0)),
        ],
        out_specs=pl.BlockSpec((BN_TC, 128), lambda i: (i, 0)),
        out_shape=jax.ShapeDtypeStruct((N_NODES, 128), jnp.float32),
    )(Sp, cnt, W2, b2.reshape(1, 256), sc2.reshape(1, 256),
      sh2.reshape(1, 256), Wout, bout.reshape(1, 128))


def _tc_edge_prep(src, N, e_per, NC, CHUNK):
    W = NC * CHUNK
    pad_per = W - e_per

    def body(e_ref, so_ref, do_ref):
        so_ref[:, :e_per] = e_ref[0]
        do_ref[:, :e_per] = e_ref[1]
        so_ref[:, e_per:] = jnp.zeros((8, pad_per), jnp.int32)
        do_ref[:, e_per:] = N + lax.broadcasted_iota(jnp.int32, (8, pad_per), 1)

    so, do = pl.pallas_call(
        body, grid=(NW // 8,),
        in_specs=[pl.BlockSpec((2, 8, e_per), lambda i: (0, i, 0))],
        out_specs=[pl.BlockSpec((8, W), lambda i: (i, 0)),
                   pl.BlockSpec((8, W), lambda i: (i, 0))],
        out_shape=[jax.ShapeDtypeStruct((NW, W), jnp.int32),
                   jax.ShapeDtypeStruct((NW, W), jnp.int32)],
    )(src.reshape(2, NW, e_per))
    return so.reshape(NW, NC, CHUNK), do.reshape(NW, NC, CHUNK)


def kernel(x, edge_index, batch, W1, b1, g1, be1, rm1, rv1, Wg, bg, gg, beg,
           rmg, rvg, W2, b2, g2, be2, rm2, rv2, Wout, bout):
    del batch
    N = x.shape[0]
    E = edge_index.shape[1]
    src = edge_index[0]
    dst = edge_index[1]

    e_per = E // NW
    src_r, dst_r = _tc_edge_prep(edge_index, N, e_per, 80, 128)
    src_r2, dst_r2 = _tc_edge_prep(edge_index, N, e_per, 90, 112)

    eps = 1e-5
    sc1 = g1 / jnp.sqrt(rv1 + eps)
    sh1 = be1 - rm1 * sc1
    scg = gg / jnp.sqrt(rvg + eps)
    shg = beg - rmg * scg
    sc2 = g2 / jnp.sqrt(rv2 + eps)
    sh2 = be2 - rm2 * sc2

    x_rep = jnp.broadcast_to(x[None], (8, N, 128)) + jnp.zeros(
        (8, 1, 1), jnp.float32)

    S1p, C1p = _make_sc_agg(128, with_counts=True)(x_rep, src_r, dst_r)
    h1, cnt = _tc_layer(S1p, C1p, W1, b1, sc1, sh1, first=True)

    S2p = _make_sc_agg(128, CHUNK=112, NC=90, NB=3)(h1, src_r2, dst_r2)
    h2 = _tc_layer(S2p, cnt, Wg, bg, scg, shg, first=False)

    S3p = _make_sc_agg(128, CHUNK=112, NC=90, NB=3)(h2, src_r2, dst_r2)
    out = _tc_final(S3p, cnt, W2, b2, sc2, sh2, Wout, bout)

    l1_reg = jnp.array(0.0, dtype=jnp.float32)
    return (out, l1_reg)


# speedup vs baseline: 1.6326x; 1.0927x over previous
import functools

import jax
import jax.numpy as jnp
from jax import lax
from jax.experimental import pallas as pl
from jax.experimental.pallas import tpu as pltpu
from jax.experimental.pallas import tpu_sc as plsc

N_NODES = 10000
N_ACC = 10240
NW = 32
CHUNK = 128
NC = 80
E_PAD = NW * NC * CHUNK
NB = 2
BN_TC = 1000


def _make_sc_agg(D, with_counts=False, CHUNK=128, NC=80, NB=2):
    mesh = plsc.VectorSubcoreMesh(core_axis_name="c", subcore_axis_name="s")
    NI = 2 * NB

    out_type = [jax.ShapeDtypeStruct((2, N_NODES, D), jnp.float32)]
    scratch = [
        pltpu.VMEM((NI, CHUNK), jnp.int32),
        pltpu.VMEM((NI, CHUNK), jnp.int32),
        pltpu.VMEM((NB, CHUNK, D), jnp.float32),
        pltpu.VMEM_SHARED((N_ACC, D), jnp.float32),
    ]
    if with_counts:
        out_type.append(jax.ShapeDtypeStruct((2, N_NODES, 16), jnp.float32))
        scratch += [
            pltpu.VMEM((CHUNK, 16), jnp.float32),
            pltpu.VMEM((CHUNK, 16), jnp.float32),
            pltpu.VMEM_SHARED((N_ACC, 16), jnp.float32),
        ]

    @functools.partial(
        pl.kernel,
        out_type=out_type,
        mesh=mesh,
        scratch_types=scratch + [pltpu.SemaphoreType.DMA] * (NI + NI + NB),
        compiler_params=pltpu.CompilerParams(use_tc_tiling_on_sc=False),
    )
    def k(table, src_r, dst_r, *rest):
        if with_counts:
            (out, outc, sidx, didx, rows, acc, cones, cbuf, cacc) = rest[:9]
            sems = rest[9:]
        else:
            (out, sidx, didx, rows, acc) = rest[:5]
            sems = rest[5:]
        sisem = sems[:NI]
        disem = sems[NI:2 * NI]
        gsem = sems[2 * NI:]
        c = lax.axis_index("c")
        s = lax.axis_index("s")
        wid = c * 16 + s
        tab = table.at[c * 4 + s // 4]

        zvec = jnp.zeros((16,), jnp.float32)

        def zrow(i, _):
            for j in range(D // 16):
                rows[0, i, pl.ds(j * 16, 16)] = zvec
            return 0

        lax.fori_loop(0, CHUNK, zrow, 0)
        if with_counts:
            ovec = jnp.ones((16,), jnp.float32)

            def crow(i, _):
                cbuf[i, pl.ds(0, 16)] = zvec
                cones[i, pl.ds(0, 16)] = ovec
                return 0

            lax.fori_loop(0, CHUNK, crow, 0)
        ZC = 128 if CHUNK >= 128 else 80
        for z in range(640 // ZC):
            pltpu.sync_copy(rows.at[0, pl.ds(0, ZC)],
                            acc.at[pl.ds(s * 640 + z * ZC, ZC)])
            if with_counts:
                pltpu.sync_copy(cbuf.at[pl.ds(0, ZC)],
                                cacc.at[pl.ds(s * 640 + z * ZC, ZC)])
        plsc.subcore_barrier()

        def fire_idx(j, sl):
            pltpu.async_copy(src_r.at[wid, j], sidx.at[sl], sisem[sl])
            pltpu.async_copy(dst_r.at[wid, j], didx.at[sl], disem[sl])

        def fire_gather(j, sl, rsl):
            pltpu.make_async_copy(
                src_r.at[0, 0], sidx.at[sl], sisem[sl]).wait()
            pltpu.async_copy(tab.at[sidx.at[sl]], rows.at[rsl], gsem[rsl])

        def do_scatter(j, sl, rsl):
            pltpu.make_async_copy(
                tab.at[sidx.at[0]], rows.at[rsl], gsem[rsl]).wait()
            pltpu.make_async_copy(
                dst_r.at[0, 0], didx.at[sl], disem[sl]).wait()
            pltpu.sync_copy(rows.at[rsl], acc.at[didx.at[sl]], add=True)
            if with_counts:
                pltpu.sync_copy(cones, cacc.at[didx.at[sl]], add=True)

        for i in range(2 * NB):
            if i >= NB:
                fire_gather(i - NB, (i - NB) % NI, (i - NB) % NB)
            fire_idx(i, i % NI)

        def body(g, _):
            for u in range(2 * NB):
                i = 2 * NB + g * 2 * NB + u
                do_scatter(i - 2 * NB, u, u % NB)
                fire_gather(i - NB, (u + NB) % NI, u % NB)
                fire_idx(i, u)
            return 0

        lax.fori_loop(0, (NC - 2 * NB) // (2 * NB), body, 0)

        for i in range(NC, NC + 2 * NB):
            do_scatter(i - 2 * NB, (i - 2 * NB) % NI, (i - 2 * NB) % NB)
            if i - NB < NC:
                fire_gather(i - NB, (i - NB) % NI, (i - NB) % NB)
        plsc.subcore_barrier()

        sizes = []
        rem = 625
        while rem:
            t = min(rem, CHUNK if CHUNK < 125 else 125)
            sizes.append(t)
            rem -= t
        r0 = 0
        for t in sizes:
            ro = s * 625 + r0
            pltpu.sync_copy(acc.at[pl.ds(ro, t)], rows.at[0, pl.ds(0, t)])
            pltpu.sync_copy(rows.at[0, pl.ds(0, t)], out.at[c, pl.ds(ro, t)])
            if with_counts:
                pltpu.sync_copy(cacc.at[pl.ds(ro, t)], cbuf.at[pl.ds(0, t)])
                pltpu.sync_copy(
                    cbuf.at[pl.ds(0, t)], outc.at[c, pl.ds(ro, t)])
            r0 += t

    if with_counts:
        return k
    return lambda *a: k(*a)[0]


def _tc_layer(Sp, cnt, W, b, sc, sh, first):
    D = Sp.shape[-1]
    H = W.shape[0]
    grid = (N_NODES // BN_TC,)

    def body(*refs):
        if first:
            p_ref, c_in_ref, w_ref, b_ref, sc_ref, sh_ref, h_ref, c_ref = refs
        else:
            p_ref, c_in_ref, w_ref, b_ref, sc_ref, sh_ref, h_ref = refs
        S = p_ref[0] + p_ref[1]
        if first:
            cc = (c_in_ref[0] + c_in_ref[1])[:, 0:1]
        else:
            cc = c_in_ref[:, 0:1]
        r = jnp.where(cc > 0, 1.0 / jnp.maximum(cc, 1.0), 0.0)
        A = S * r
        Z = lax.dot_general(A, w_ref[...], (((1,), (1,)), ((), ())),
                            preferred_element_type=jnp.float32)
        Z = jnp.where(cc > 0, Z + b_ref[...], 0.0)
        Z = Z * sc_ref[...] + sh_ref[...]
        h = jnp.where(Z > 0, Z, jnp.exp(Z) - 1.0)
        for _r in range(8):
            h_ref[_r] = h
        if first:
            c_ref[...] = jnp.broadcast_to(cc, (BN_TC, 8))

    in_specs = [pl.BlockSpec((2, BN_TC, D), lambda i: (0, i, 0))]
    if first:
        in_specs.append(pl.BlockSpec((2, BN_TC, 16), lambda i: (0, i, 0)))
    else:
        in_specs.append(pl.BlockSpec((BN_TC, 8), lambda i: (i, 0)))
    in_specs += [
        pl.BlockSpec(W.shape, lambda i: (0, 0)),
        pl.BlockSpec((1, H), lambda i: (0, 0)),
        pl.BlockSpec((1, H), lambda i: (0, 0)),
        pl.BlockSpec((1, H), lambda i: (0, 0)),
    ]
    out_shape = [jax.ShapeDtypeStruct((8, N_NODES, H), jnp.float32)]
    out_specs = [pl.BlockSpec((8, BN_TC, H), lambda i: (0, i, 0))]
    if first:
        out_shape.append(jax.ShapeDtypeStruct((N_NODES, 8), jnp.float32))
        out_specs.append(pl.BlockSpec((BN_TC, 8), lambda i: (i, 0)))

    args = [Sp, cnt]
    args += [W, b.reshape(1, H), sc.reshape(1, H), sh.reshape(1, H)]
    res = pl.pallas_call(
        body, grid=grid, in_specs=in_specs, out_specs=out_specs,
        out_shape=out_shape)(*args)
    return res if first else res[0]


def _tc_final(Sp, cnt, W2, b2, sc2, sh2, Wout, bout):
    D = Sp.shape[-1]
    grid = (N_NODES // BN_TC,)

    def body(p_ref, c_ref, w2_ref, b2_ref, sc_ref, sh_ref, wo_ref, bo_ref,
             o_ref):
        P = p_ref[0] + p_ref[1]
        cc = c_ref[:, 0:1]
        r = jnp.where(cc > 0, 1.0 / jnp.maximum(cc, 1.0), 0.0)
        A = P * r
        Z = lax.dot_general(A, w2_ref[...], (((1,), (1,)), ((), ())),
                            preferred_element_type=jnp.float32)
        Z = jnp.where(cc > 0, Z + b2_ref[...], 0.0)
        Z = Z * sc_ref[...] + sh_ref[...]
        h3 = jnp.where(Z > 0, Z, jnp.exp(Z) - 1.0)
        o_ref[...] = lax.dot_general(h3, wo_ref[...], (((1,), (1,)), ((), ())),
                                     preferred_element_type=jnp.float32) + bo_ref[...]

    return pl.pallas_call(
        body, grid=grid,
        in_specs=[
            pl.BlockSpec((2, BN_TC, D), lambda i: (0, i, 0)),
            pl.BlockSpec((BN_TC, 8), lambda i: (i, 0)),
            pl.BlockSpec(W2.shape, lambda i: (0, 0)),
            pl.BlockSpec((1, 256), lambda i: (0, 0)),
            pl.BlockSpec((1, 256), lambda i: (0, 0)),
            pl.BlockSpec((1, 256), lambda i: (0, 0)),
            pl.BlockSpec(Wout.shape, lambda i: (0, 0)),
            pl.BlockSpec((1, 128), lambda i: (0, 0)),
        ],
        out_specs=pl.BlockSpec((BN_TC, 128), lambda i: (i, 0)),
        out_shape=jax.ShapeDtypeStruct((N_NODES, 128), jnp.float32),
    )(Sp, cnt, W2, b2.reshape(1, 256), sc2.reshape(1, 256),
      sh2.reshape(1, 256), Wout, bout.reshape(1, 128))


def _tc_edge_prep(src, N, e_per, NC, CHUNK):
    W = NC * CHUNK
    pad_per = W - e_per

    def body(e_ref, so_ref, do_ref):
        so_ref[:, :e_per] = e_ref[0]
        do_ref[:, :e_per] = e_ref[1]
        so_ref[:, e_per:] = jnp.zeros((8, pad_per), jnp.int32)
        do_ref[:, e_per:] = N + lax.broadcasted_iota(jnp.int32, (8, pad_per), 1)

    so, do = pl.pallas_call(
        body, grid=(NW // 8,),
        in_specs=[pl.BlockSpec((2, 8, e_per), lambda i: (0, i, 0))],
        out_specs=[pl.BlockSpec((8, W), lambda i: (i, 0)),
                   pl.BlockSpec((8, W), lambda i: (i, 0))],
        out_shape=[jax.ShapeDtypeStruct((NW, W), jnp.int32),
                   jax.ShapeDtypeStruct((NW, W), jnp.int32)],
    )(src.reshape(2, NW, e_per))
    return so.reshape(NW, NC, CHUNK), do.reshape(NW, NC, CHUNK)


def kernel(x, edge_index, batch, W1, b1, g1, be1, rm1, rv1, Wg, bg, gg, beg,
           rmg, rvg, W2, b2, g2, be2, rm2, rv2, Wout, bout):
    del batch
    N = x.shape[0]
    E = edge_index.shape[1]
    src = edge_index[0]
    dst = edge_index[1]

    e_per = E // NW
    src_r, dst_r = _tc_edge_prep(edge_index, N, e_per, 114, 88)
    src_r2, dst_r2 = _tc_edge_prep(edge_index, N, e_per, 90, 112)

    eps = 1e-5
    sc1 = g1 / jnp.sqrt(rv1 + eps)
    sh1 = be1 - rm1 * sc1
    scg = gg / jnp.sqrt(rvg + eps)
    shg = beg - rmg * scg
    sc2 = g2 / jnp.sqrt(rv2 + eps)
    sh2 = be2 - rm2 * sc2

    x_rep = jnp.broadcast_to(x[None], (8, N, 128)) + jnp.zeros(
        (8, 1, 1), jnp.float32)

    S1p, C1p = _make_sc_agg(128, with_counts=True, CHUNK=88, NC=114, NB=3)(x_rep, src_r, dst_r)
    h1, cnt = _tc_layer(S1p, C1p, W1, b1, sc1, sh1, first=True)

    S2p = _make_sc_agg(128, CHUNK=112, NC=90, NB=3)(h1, src_r2, dst_r2)
    h2 = _tc_layer(S2p, cnt, Wg, bg, scg, shg, first=False)

    S3p = _make_sc_agg(128, CHUNK=112, NC=90, NB=3)(h2, src_r2, dst_r2)
    out = _tc_final(S3p, cnt, W2, b2, sc2, sh2, Wout, bout)

    l1_reg = jnp.array(0.0, dtype=jnp.float32)
    return (out, l1_reg)
